# baseline (speedup 1.0000x reference)
import functools

import jax
import jax.numpy as jnp
from jax import lax
from jax.experimental import pallas as pl
from jax.experimental.pallas import tpu as pltpu
from jax.experimental.pallas import tpu_sc as plsc

_L = 19
_F = 2
_T = 1 << 20
_BASE = 16
_N = 262144
_PRIME = 2654435761
_PRIME_I32 = _PRIME - (1 << 32)
_MASK = _T - 1
_EC = 2 * _L
_ECP = 40

_NC = 2
_NS = 16
_NW = _NC * _NS


def _build_encode(n, b):
    pw = n // _NW
    nsub = pw // b
    ng = b // 16

    mesh = plsc.VectorSubcoreMesh(core_axis_name="c", subcore_axis_name="s")

    @functools.partial(
        pl.kernel,
        out_type=jax.ShapeDtypeStruct((n * _ECP,), jnp.float32),
        mesh=mesh,
        compiler_params=pltpu.CompilerParams(
            needs_layout_passes=False, use_tc_tiling_on_sc=False),
        scratch_types=[
            pltpu.VMEM((b,), jnp.float32),
            pltpu.VMEM((b,), jnp.float32),
            pltpu.VMEM((b,), jnp.float32),
            pltpu.VMEM((b,), jnp.float32),
            pltpu.VMEM((b,), jnp.int32),
            pltpu.VMEM((b,), jnp.int32),
            pltpu.VMEM((b,), jnp.int32),
            pltpu.VMEM((b,), jnp.int32),
            pltpu.VMEM((b,), jnp.int32),
            pltpu.VMEM((b,), jnp.int32),
            pltpu.VMEM((b,), jnp.int32),
            pltpu.VMEM((b,), jnp.int32),
            pltpu.VMEM((b, 8), jnp.float32),
            pltpu.VMEM((b, 8), jnp.float32),
            pltpu.VMEM((b, 8), jnp.float32),
            pltpu.VMEM((b, 8), jnp.float32),
            pltpu.VMEM((b * _ECP,), jnp.float32),
            pltpu.SemaphoreType.DMA,
        ],
    )
    def encode(x0_hbm, x1_hbm, grid_hbm, out_hbm,
               x0b, x1b, f0b, f1b, ib0, ib1, ib2, ib3,
               lo0, lo1, lo2, lo3, r0, r1, r2, r3, encb, sem):
        wid = lax.axis_index("s") * _NC + lax.axis_index("c")

        def sub_body(sub, carry):
            gbase = wid * pw + sub * b
            pltpu.sync_copy(x0_hbm.at[pl.ds(gbase, b)], x0b)
            pltpu.sync_copy(x1_hbm.at[pl.ds(gbase, b)], x1b)

            io = lax.iota(jnp.int32, 16)

            def pz(j, c):
                eidx = (j * 16 + io) * _ECP
                z16 = jnp.zeros((16,), jnp.float32)
                plsc.store_scatter(encb, [eidx + _EC], z16)
                plsc.store_scatter(encb, [eidx + (_EC + 1)], z16)
                return c
            lax.fori_loop(0, ng, pz, 0)

            for l in range(_L):
                scale = _BASE * (2 ** l) - 1
                res = scale + 1
                dense = (res * res) <= _T
                off4 = l * (_T // 4)
                scale_f = float(scale)

                def pa(j, c, _scale_f=scale_f, _scale=scale, _res=res,
                       _dense=dense, _off4=off4):
                    sv = j * 16
                    xv0 = x0b[pl.ds(sv, 16)]
                    xv1 = x1b[pl.ds(sv, 16)]
                    pos0 = xv0 * _scale_f
                    pos1 = xv1 * _scale_f
                    i0 = jnp.minimum(pos0.astype(jnp.int32), _scale - 1)
                    i1 = jnp.minimum(pos1.astype(jnp.int32), _scale - 1)
                    f0b[pl.ds(sv, 16)] = pos0 - i0.astype(jnp.float32)
                    f1b[pl.ds(sv, 16)] = pos1 - i1.astype(jnp.float32)
                    cx1 = i0 + 1
                    cy1 = i1 + 1
                    if _dense:
                        idx00 = i0 * _res + i1
                        idx01 = i0 * _res + cy1
                        idx10 = cx1 * _res + i1
                        idx11 = cx1 * _res + cy1
                    else:
                        t0 = i1 * _PRIME_I32
                        t1 = cy1 * _PRIME_I32
                        idx00 = (i0 ^ t0) & _MASK
                        idx01 = (i0 ^ t1) & _MASK
                        idx10 = (cx1 ^ t0) & _MASK
                        idx11 = (cx1 ^ t1) & _MASK
                    ib0[pl.ds(sv, 16)] = (idx00 >> 2) + _off4
                    ib1[pl.ds(sv, 16)] = (idx01 >> 2) + _off4
                    ib2[pl.ds(sv, 16)] = (idx10 >> 2) + _off4
                    ib3[pl.ds(sv, 16)] = (idx11 >> 2) + _off4
                    lo0[pl.ds(sv, 16)] = (idx00 & 3) << 1
                    lo1[pl.ds(sv, 16)] = (idx01 & 3) << 1
                    lo2[pl.ds(sv, 16)] = (idx10 & 3) << 1
                    lo3[pl.ds(sv, 16)] = (idx11 & 3) << 1
                    return c
                lax.fori_loop(0, ng, pa, 0)

                cps = [pltpu.async_copy(grid_hbm.at[ib], rb, sem)
                       for ib, rb in ((ib0, r0), (ib1, r1), (ib2, r2), (ib3, r3))]
                for cp in cps:
                    cp.wait()

                def pc(j, c, _l=l):
                    sv = j * 16
                    pidx = sv + io
                    c0 = lo0[pl.ds(sv, 16)]
                    c1 = lo1[pl.ds(sv, 16)]
                    c2 = lo2[pl.ds(sv, 16)]
                    c3 = lo3[pl.ds(sv, 16)]
                    f0 = f0b[pl.ds(sv, 16)]
                    f1 = f1b[pl.ds(sv, 16)]
                    g0 = 1.0 - f0
                    g1 = 1.0 - f1
                    w00 = g0 * g1
                    w01 = g0 * f1
                    w10 = f0 * g1
                    w11 = f0 * f1
                    a0 = (w00 * plsc.load_gather(r0, [pidx, c0])
                          + w01 * plsc.load_gather(r1, [pidx, c1])
                          + w10 * plsc.load_gather(r2, [pidx, c2])
                          + w11 * plsc.load_gather(r3, [pidx, c3]))
                    a1 = (w00 * plsc.load_gather(r0, [pidx, c0 + 1])
                          + w01 * plsc.load_gather(r1, [pidx, c1 + 1])
                          + w10 * plsc.load_gather(r2, [pidx, c2 + 1])
                          + w11 * plsc.load_gather(r3, [pidx, c3 + 1]))
                    eidx = pidx * _ECP
                    plsc.store_scatter(encb, [eidx + 2 * _l], a0)
                    plsc.store_scatter(encb, [eidx + (2 * _l + 1)], a1)
                    return c
                lax.fori_loop(0, ng, pc, 0)

            pltpu.sync_copy(encb, out_hbm.at[pl.ds(gbase * _ECP, b * _ECP)])
            return carry

        lax.fori_loop(0, nsub, sub_body, 0)

    return encode


_encode = _build_encode(_N, 1024)

_MLP_NB = 2048


def _mlp_body(enc_ref, w1_ref, w2_ref, w3_ref, out_ref):
    h = jnp.maximum(
        jnp.dot(enc_ref[...], w1_ref[...], preferred_element_type=jnp.float32), 0.0)
    h = jnp.maximum(
        jnp.dot(h, w2_ref[...], preferred_element_type=jnp.float32), 0.0)
    out_ref[...] = jnp.dot(h, w3_ref[...], preferred_element_type=jnp.float32)


def _mlp(enc, w1p, w2, w3):
    n = enc.shape[0]
    return pl.pallas_call(
        _mlp_body,
        grid=(n // _MLP_NB,),
        in_specs=[
            pl.BlockSpec((_MLP_NB, _ECP), lambda i: (i, 0)),
            pl.BlockSpec((_ECP, 64), lambda i: (0, 0)),
            pl.BlockSpec((64, 64), lambda i: (0, 0)),
            pl.BlockSpec((64, 3), lambda i: (0, 0)),
        ],
        out_specs=pl.BlockSpec((_MLP_NB, 3), lambda i: (i, 0)),
        out_shape=jax.ShapeDtypeStruct((n, 3), jnp.float32),
    )(enc, w1p, w2, w3)


def kernel(x, grid, W1, W2, W3):
    x0 = jnp.asarray(x[:, 0])
    x1 = jnp.asarray(x[:, 1])
    gridf = grid.reshape(_L * _T // 4, 4 * _F)
    enc = _encode(x0, x1, gridf).reshape(_N, _ECP)
    w1p = jnp.concatenate([W1, jnp.zeros((2, 64), jnp.float32)], axis=0)
    return _mlp(enc, w1p, W2, W3)


# speedup vs baseline: 7.0210x; 7.0210x over previous
import functools

import jax
import jax.numpy as jnp
from jax import lax
from jax.experimental import pallas as pl
from jax.experimental.pallas import tpu as pltpu
from jax.experimental.pallas import tpu_sc as plsc

_L = 19
_F = 2
_T = 1 << 20
_BASE = 16
_N = 262144
_PRIME = 2654435761
_PRIME_I32 = _PRIME - (1 << 32)
_MASK = _T - 1
_EC = 2 * _L
_ECP = 40

_NC = 2
_NS = 16
_NW = _NC * _NS


def _build_encode(n, b):
    pw = n // _NW
    nsub = pw // b
    ng = b // 16

    mesh = plsc.VectorSubcoreMesh(core_axis_name="c", subcore_axis_name="s")

    @functools.partial(
        pl.kernel,
        out_type=jax.ShapeDtypeStruct((n * _ECP,), jnp.float32),
        mesh=mesh,
        compiler_params=pltpu.CompilerParams(
            needs_layout_passes=False, use_tc_tiling_on_sc=False),
        scratch_types=[
            pltpu.VMEM((b,), jnp.float32),
            pltpu.VMEM((b,), jnp.float32),
            pltpu.VMEM((b,), jnp.float32),
            pltpu.VMEM((b,), jnp.float32),
            pltpu.VMEM((2 * b,), jnp.int32),
            pltpu.VMEM((2 * b,), jnp.int32),
            pltpu.VMEM((2 * b,), jnp.int32),
            pltpu.VMEM((2 * b,), jnp.int32),
            pltpu.VMEM((b,), jnp.int32),
            pltpu.VMEM((b,), jnp.int32),
            pltpu.VMEM((b,), jnp.int32),
            pltpu.VMEM((b,), jnp.int32),
            pltpu.VMEM((2 * b, 8), jnp.float32),
            pltpu.VMEM((2 * b, 8), jnp.float32),
            pltpu.VMEM((2 * b, 8), jnp.float32),
            pltpu.VMEM((2 * b, 8), jnp.float32),
            pltpu.VMEM((b * _ECP,), jnp.float32),
            pltpu.SemaphoreType.DMA,
        ],
    )
    def encode(x0_hbm, x1_hbm, grid_hbm, out_hbm,
               x0b, x1b, f0b, f1b, ib0, ib1, ib2, ib3,
               lo0, lo1, lo2, lo3, r0, r1, r2, r3, encb, sem):
        wid = lax.axis_index("s") * _NC + lax.axis_index("c")

        def sub_body(sub, carry):
            gbase = wid * pw + sub * b
            pltpu.sync_copy(x0_hbm.at[pl.ds(gbase, b)], x0b)
            pltpu.sync_copy(x1_hbm.at[pl.ds(gbase, b)], x1b)

            io = lax.iota(jnp.int32, 16)

            def pz(j, c):
                eidx = (j * 16 + io) * _ECP
                z16 = jnp.zeros((16,), jnp.float32)
                plsc.store_scatter(encb, [eidx + _EC], z16)
                plsc.store_scatter(encb, [eidx + (_EC + 1)], z16)
                return c
            lax.fori_loop(0, ng, pz, 0)

            for l in range(_L):
                scale = _BASE * (2 ** l) - 1
                res = scale + 1
                dense = (res * res) <= _T
                loff = l * (_T // 4)
                scale_f = float(scale)

                def pa(j, c, _scale_f=scale_f, _scale=scale, _res=res,
                       _dense=dense, _loff=loff):
                    sv = j * 16
                    e2 = 2 * (sv + io)
                    xv0 = x0b[pl.ds(sv, 16)]
                    xv1 = x1b[pl.ds(sv, 16)]
                    pos0 = xv0 * _scale_f
                    pos1 = xv1 * _scale_f
                    i0 = jnp.minimum(pos0.astype(jnp.int32), _scale - 1)
                    i1 = jnp.minimum(pos1.astype(jnp.int32), _scale - 1)
                    f0b[pl.ds(sv, 16)] = pos0 - i0.astype(jnp.float32)
                    f1b[pl.ds(sv, 16)] = pos1 - i1.astype(jnp.float32)
                    cx1 = i0 + 1
                    cy1 = i1 + 1
                    if _dense:
                        idx00 = i0 * _res + i1
                        idx01 = i0 * _res + cy1
                        idx10 = cx1 * _res + i1
                        idx11 = cx1 * _res + cy1
                    else:
                        t0 = i1 * _PRIME_I32
                        t1 = cy1 * _PRIME_I32
                        idx00 = (i0 ^ t0) & _MASK
                        idx01 = (i0 ^ t1) & _MASK
                        idx10 = (cx1 ^ t0) & _MASK
                        idx11 = (cx1 ^ t1) & _MASK
                    r00 = _loff + ((idx00 >> 7) << 5) + ((idx00 >> 3) & 15)
                    r01 = _loff + ((idx01 >> 7) << 5) + ((idx01 >> 3) & 15)
                    r10 = _loff + ((idx10 >> 7) << 5) + ((idx10 >> 3) & 15)
                    r11 = _loff + ((idx11 >> 7) << 5) + ((idx11 >> 3) & 15)
                    plsc.store_scatter(ib0, [e2], r00)
                    plsc.store_scatter(ib0, [e2 + 1], r00 + 16)
                    plsc.store_scatter(ib1, [e2], r01)
                    plsc.store_scatter(ib1, [e2 + 1], r01 + 16)
                    plsc.store_scatter(ib2, [e2], r10)
                    plsc.store_scatter(ib2, [e2 + 1], r10 + 16)
                    plsc.store_scatter(ib3, [e2], r11)
                    plsc.store_scatter(ib3, [e2 + 1], r11 + 16)
                    lo0[pl.ds(sv, 16)] = idx00 & 7
                    lo1[pl.ds(sv, 16)] = idx01 & 7
                    lo2[pl.ds(sv, 16)] = idx10 & 7
                    lo3[pl.ds(sv, 16)] = idx11 & 7
                    return c
                lax.fori_loop(0, ng, pa, 0)

                cps = [pltpu.async_copy(grid_hbm.at[ib], rb, sem)
                       for ib, rb in ((ib0, r0), (ib1, r1), (ib2, r2), (ib3, r3))]
                for cp in cps:
                    cp.wait()

                def pc(j, c, _l=l):
                    sv = j * 16
                    pidx = sv + io
                    e2 = 2 * pidx
                    c0 = lo0[pl.ds(sv, 16)]
                    c1 = lo1[pl.ds(sv, 16)]
                    c2 = lo2[pl.ds(sv, 16)]
                    c3 = lo3[pl.ds(sv, 16)]
                    f0 = f0b[pl.ds(sv, 16)]
                    f1 = f1b[pl.ds(sv, 16)]
                    g0 = 1.0 - f0
                    g1 = 1.0 - f1
                    w00 = g0 * g1
                    w01 = g0 * f1
                    w10 = f0 * g1
                    w11 = f0 * f1
                    a0 = (w00 * plsc.load_gather(r0, [e2, c0])
                          + w01 * plsc.load_gather(r1, [e2, c1])
                          + w10 * plsc.load_gather(r2, [e2, c2])
                          + w11 * plsc.load_gather(r3, [e2, c3]))
                    a1 = (w00 * plsc.load_gather(r0, [e2 + 1, c0])
                          + w01 * plsc.load_gather(r1, [e2 + 1, c1])
                          + w10 * plsc.load_gather(r2, [e2 + 1, c2])
                          + w11 * plsc.load_gather(r3, [e2 + 1, c3]))
                    eidx = pidx * _ECP
                    plsc.store_scatter(encb, [eidx + 2 * _l], a0)
                    plsc.store_scatter(encb, [eidx + (2 * _l + 1)], a1)
                    return c
                lax.fori_loop(0, ng, pc, 0)

            pltpu.sync_copy(encb, out_hbm.at[pl.ds(gbase * _ECP, b * _ECP)])
            return carry

        lax.fori_loop(0, nsub, sub_body, 0)

    return encode


_encode = _build_encode(_N, 512)

_MLP_NB = 2048


def _mlp_body(enc_ref, w1_ref, w2_ref, w3_ref, out_ref):
    h = jnp.maximum(
        jnp.dot(enc_ref[...], w1_ref[...], preferred_element_type=jnp.float32), 0.0)
    h = jnp.maximum(
        jnp.dot(h, w2_ref[...], preferred_element_type=jnp.float32), 0.0)
    out_ref[...] = jnp.dot(h, w3_ref[...], preferred_element_type=jnp.float32)


def _mlp(enc, w1p, w2, w3):
    n = enc.shape[0]
    return pl.pallas_call(
        _mlp_body,
        grid=(n // _MLP_NB,),
        in_specs=[
            pl.BlockSpec((_MLP_NB, _ECP), lambda i: (i, 0)),
            pl.BlockSpec((_ECP, 64), lambda i: (0, 0)),
            pl.BlockSpec((64, 64), lambda i: (0, 0)),
            pl.BlockSpec((64, 3), lambda i: (0, 0)),
        ],
        out_specs=pl.BlockSpec((_MLP_NB, 3), lambda i: (i, 0)),
        out_shape=jax.ShapeDtypeStruct((n, 3), jnp.float32),
    )(enc, w1p, w2, w3)


def kernel(x, grid, W1, W2, W3):
    x0 = jnp.asarray(x[:, 0])
    x1 = jnp.asarray(x[:, 1])
    gridf = (grid.reshape(_L, _T // 128, 128, _F)
             .transpose(0, 1, 3, 2)
             .reshape(_L * _T // 4, 4 * _F))
    enc = _encode(x0, x1, gridf).reshape(_N, _ECP)
    w1p = jnp.concatenate([W1, jnp.zeros((2, 64), jnp.float32)], axis=0)
    return _mlp(enc, w1p, W2, W3)


# speedup vs baseline: 7.4100x; 1.0554x over previous
import functools

import jax
import jax.numpy as jnp
from jax import lax
from jax.experimental import pallas as pl
from jax.experimental.pallas import tpu as pltpu
from jax.experimental.pallas import tpu_sc as plsc

_L = 19
_F = 2
_T = 1 << 20
_BASE = 16
_N = 262144
_PRIME = 2654435761
_PRIME_I32 = _PRIME - (1 << 32)
_MASK = _T - 1
_EC = 2 * _L
_ECP = 40

_NC = 2
_NS = 16
_NW = _NC * _NS


def _build_encode(n, b):
    pw = n // _NW
    nsub = pw // b
    ng = b // 16

    mesh = plsc.VectorSubcoreMesh(core_axis_name="c", subcore_axis_name="s")

    scratch = [
        pltpu.VMEM((b,), jnp.float32),
        pltpu.VMEM((b,), jnp.float32),
        pltpu.VMEM((b * _ECP,), jnp.float32),
    ]
    for _set in range(2):
        scratch += [pltpu.VMEM((b,), jnp.float32)] * 2
        scratch += [pltpu.VMEM((b,), jnp.int32)] * 4
        scratch += [pltpu.VMEM((2 * b,), jnp.int32)] * 4
        scratch += [pltpu.VMEM((2 * b, 8), jnp.float32)] * 4
        scratch += [pltpu.SemaphoreType.DMA]

    @functools.partial(
        pl.kernel,
        out_type=jax.ShapeDtypeStruct((n * _ECP,), jnp.float32),
        mesh=mesh,
        compiler_params=pltpu.CompilerParams(
            needs_layout_passes=False, use_tc_tiling_on_sc=False),
        scratch_types=scratch,
    )
    def encode(x0_hbm, x1_hbm, grid_hbm, out_hbm, x0b, x1b, encb, *bufs):
        sets = []
        for _s in range(2):
            o = _s * 15
            sets.append(dict(
                fb=bufs[o:o + 2], lo=bufs[o + 2:o + 6],
                ib=bufs[o + 6:o + 10], rb=bufs[o + 10:o + 14],
                sem=bufs[o + 14]))

        wid = lax.axis_index("s") * _NC + lax.axis_index("c")
        io = lax.iota(jnp.int32, 16)

        def do_pa(l, st):
            scale = _BASE * (2 ** l) - 1
            res = scale + 1
            dense = (res * res) <= _T
            loff = l * (_T // 4)
            scale_f = float(scale)
            fb, lo, ib = st["fb"], st["lo"], st["ib"]

            def pa(j, c):
                sv = j * 16
                e2 = 2 * (sv + io)
                xv0 = x0b[pl.ds(sv, 16)]
                xv1 = x1b[pl.ds(sv, 16)]
                pos0 = xv0 * scale_f
                pos1 = xv1 * scale_f
                i0 = jnp.minimum(pos0.astype(jnp.int32), scale - 1)
                i1 = jnp.minimum(pos1.astype(jnp.int32), scale - 1)
                fb[0][pl.ds(sv, 16)] = pos0 - i0.astype(jnp.float32)
                fb[1][pl.ds(sv, 16)] = pos1 - i1.astype(jnp.float32)
                cx1 = i0 + 1
                cy1 = i1 + 1
                if dense:
                    idx = (i0 * res + i1, i0 * res + cy1,
                           cx1 * res + i1, cx1 * res + cy1)
                else:
                    t0 = i1 * _PRIME_I32
                    t1 = cy1 * _PRIME_I32
                    idx = ((i0 ^ t0) & _MASK, (i0 ^ t1) & _MASK,
                           (cx1 ^ t0) & _MASK, (cx1 ^ t1) & _MASK)
                for c_ in range(4):
                    t = idx[c_]
                    r_ = loff + ((t >> 7) << 5) + ((t >> 3) & 15)
                    plsc.store_scatter(ib[c_], [e2], r_)
                    plsc.store_scatter(ib[c_], [e2 + 1], r_ + 16)
                    lo[c_][pl.ds(sv, 16)] = t & 7
                return c
            lax.fori_loop(0, ng, pa, 0)

        def do_fire(st):
            return [pltpu.async_copy(grid_hbm.at[ib_], rb_, st["sem"])
                    for ib_, rb_ in zip(st["ib"], st["rb"])]

        def do_pc(l, st):
            fb, lo, rb = st["fb"], st["lo"], st["rb"]

            def pc(j, c):
                sv = j * 16
                pidx = sv + io
                e2 = 2 * pidx
                cc = [lo[c_][pl.ds(sv, 16)] for c_ in range(4)]
                f0 = fb[0][pl.ds(sv, 16)]
                f1 = fb[1][pl.ds(sv, 16)]
                g0 = 1.0 - f0
                g1 = 1.0 - f1
                w = (g0 * g1, g0 * f1, f0 * g1, f0 * f1)
                a0 = (w[0] * plsc.load_gather(rb[0], [e2, cc[0]])
                      + w[1] * plsc.load_gather(rb[1], [e2, cc[1]])
                      + w[2] * plsc.load_gather(rb[2], [e2, cc[2]])
                      + w[3] * plsc.load_gather(rb[3], [e2, cc[3]]))
                a1 = (w[0] * plsc.load_gather(rb[0], [e2 + 1, cc[0]])
                      + w[1] * plsc.load_gather(rb[1], [e2 + 1, cc[1]])
                      + w[2] * plsc.load_gather(rb[2], [e2 + 1, cc[2]])
                      + w[3] * plsc.load_gather(rb[3], [e2 + 1, cc[3]]))
                eidx = pidx * _ECP
                plsc.store_scatter(encb, [eidx + 2 * l], a0)
                plsc.store_scatter(encb, [eidx + (2 * l + 1)], a1)
                return c
            lax.fori_loop(0, ng, pc, 0)

        def sub_body(sub, carry):
            gbase = wid * pw + sub * b
            pltpu.sync_copy(x0_hbm.at[pl.ds(gbase, b)], x0b)
            pltpu.sync_copy(x1_hbm.at[pl.ds(gbase, b)], x1b)

            def pz(j, c):
                eidx = (j * 16 + io) * _ECP
                z16 = jnp.zeros((16,), jnp.float32)
                plsc.store_scatter(encb, [eidx + _EC], z16)
                plsc.store_scatter(encb, [eidx + (_EC + 1)], z16)
                return c
            lax.fori_loop(0, ng, pz, 0)

            do_pa(0, sets[0])
            cps = do_fire(sets[0])
            for l in range(1, _L):
                cur = sets[l & 1]
                do_pa(l, cur)
                ncps = do_fire(cur)
                for cp in cps:
                    cp.wait()
                do_pc(l - 1, sets[(l - 1) & 1])
                cps = ncps
            for cp in cps:
                cp.wait()
            do_pc(_L - 1, sets[(_L - 1) & 1])

            pltpu.sync_copy(encb, out_hbm.at[pl.ds(gbase * _ECP, b * _ECP)])
            return carry

        lax.fori_loop(0, nsub, sub_body, 0)

    return encode


_encode = _build_encode(_N, 512)

_MLP_NB = 2048


def _mlp_body(enc_ref, w1_ref, w2_ref, w3_ref, out_ref):
    h = jnp.maximum(
        jnp.dot(enc_ref[...], w1_ref[...], preferred_element_type=jnp.float32), 0.0)
    h = jnp.maximum(
        jnp.dot(h, w2_ref[...], preferred_element_type=jnp.float32), 0.0)
    out_ref[...] = jnp.dot(h, w3_ref[...], preferred_element_type=jnp.float32)


def _mlp(enc, w1p, w2, w3):
    n = enc.shape[0]
    return pl.pallas_call(
        _mlp_body,
        grid=(n // _MLP_NB,),
        in_specs=[
            pl.BlockSpec((_MLP_NB, _ECP), lambda i: (i, 0)),
            pl.BlockSpec((_ECP, 64), lambda i: (0, 0)),
            pl.BlockSpec((64, 64), lambda i: (0, 0)),
            pl.BlockSpec((64, 3), lambda i: (0, 0)),
        ],
        out_specs=pl.BlockSpec((_MLP_NB, 3), lambda i: (i, 0)),
        out_shape=jax.ShapeDtypeStruct((n, 3), jnp.float32),
    )(enc, w1p, w2, w3)


def kernel(x, grid, W1, W2, W3):
    x0 = jnp.asarray(x[:, 0])
    x1 = jnp.asarray(x[:, 1])
    gridf = (grid.reshape(_L, _T // 128, 128, _F)
             .transpose(0, 1, 3, 2)
             .reshape(_L * _T // 4, 4 * _F))
    enc = _encode(x0, x1, gridf).reshape(_N, _ECP)
    w1p = jnp.concatenate([W1, jnp.zeros((2, 64), jnp.float32)], axis=0)
    return _mlp(enc, w1p, W2, W3)


# speedup vs baseline: 7.5069x; 1.0131x over previous
import functools

import jax
import jax.numpy as jnp
from jax import lax
from jax.experimental import pallas as pl
from jax.experimental.pallas import tpu as pltpu
from jax.experimental.pallas import tpu_sc as plsc

_L = 19
_F = 2
_T = 1 << 20
_BASE = 16
_N = 262144
_PRIME = 2654435761
_PRIME_I32 = _PRIME - (1 << 32)
_MASK = _T - 1
_EC = 2 * _L
_ECP = 40

_NC = 2
_NS = 16
_NW = _NC * _NS


def _build_encode(n, b):
    pw = n // _NW
    nsub = pw // b
    ng = b // 16

    mesh = plsc.VectorSubcoreMesh(core_axis_name="c", subcore_axis_name="s")

    scratch = [
        pltpu.VMEM((b,), jnp.float32),
        pltpu.VMEM((b,), jnp.float32),
        pltpu.VMEM((b * _ECP,), jnp.float32),
    ]
    for _set in range(2):
        scratch += [pltpu.VMEM((b,), jnp.float32)] * 2
        scratch += [pltpu.VMEM((b,), jnp.int32)] * 4
        scratch += [pltpu.VMEM((b,), jnp.int32)] * 4
        scratch += [pltpu.VMEM((b, 16), jnp.float32)] * 4
        scratch += [pltpu.SemaphoreType.DMA]

    @functools.partial(
        pl.kernel,
        out_type=jax.ShapeDtypeStruct((n * _ECP,), jnp.float32),
        mesh=mesh,
        compiler_params=pltpu.CompilerParams(
            needs_layout_passes=False, use_tc_tiling_on_sc=False),
        scratch_types=scratch,
    )
    def encode(x0_hbm, x1_hbm, grid_hbm, out_hbm, x0b, x1b, encb, *bufs):
        sets = []
        for _s in range(2):
            o = _s * 15
            sets.append(dict(
                fb=bufs[o:o + 2], lo=bufs[o + 2:o + 6],
                ib=bufs[o + 6:o + 10], rb=bufs[o + 10:o + 14],
                sem=bufs[o + 14]))

        wid = lax.axis_index("s") * _NC + lax.axis_index("c")
        io = lax.iota(jnp.int32, 16)

        def do_pa(l, st):
            scale = _BASE * (2 ** l) - 1
            res = scale + 1
            dense = (res * res) <= _T
            loff = l * (_T // 8)
            scale_f = float(scale)
            fb, lo, ib = st["fb"], st["lo"], st["ib"]

            def pa(j, c):
                sv = j * 16
                xv0 = x0b[pl.ds(sv, 16)]
                xv1 = x1b[pl.ds(sv, 16)]
                pos0 = xv0 * scale_f
                pos1 = xv1 * scale_f
                i0 = jnp.minimum(pos0.astype(jnp.int32), scale - 1)
                i1 = jnp.minimum(pos1.astype(jnp.int32), scale - 1)
                fb[0][pl.ds(sv, 16)] = pos0 - i0.astype(jnp.float32)
                fb[1][pl.ds(sv, 16)] = pos1 - i1.astype(jnp.float32)
                cx1 = i0 + 1
                cy1 = i1 + 1
                if dense:
                    idx = (i0 * res + i1, i0 * res + cy1,
                           cx1 * res + i1, cx1 * res + cy1)
                else:
                    t0 = i1 * _PRIME_I32
                    t1 = cy1 * _PRIME_I32
                    idx = ((i0 ^ t0) & _MASK, (i0 ^ t1) & _MASK,
                           (cx1 ^ t0) & _MASK, (cx1 ^ t1) & _MASK)
                for c_ in range(4):
                    t = idx[c_]
                    ib[c_][pl.ds(sv, 16)] = loff + (t >> 3)
                    lo[c_][pl.ds(sv, 16)] = t & 7
                return c
            lax.fori_loop(0, ng, pa, 0)

        def do_fire(st):
            return [pltpu.async_copy(grid_hbm.at[ib_], rb_, st["sem"])
                    for ib_, rb_ in zip(st["ib"], st["rb"])]

        def do_pc(l, st):
            fb, lo, rb = st["fb"], st["lo"], st["rb"]

            def pc(j, c):
                sv = j * 16
                pidx = sv + io
                cc = [lo[c_][pl.ds(sv, 16)] for c_ in range(4)]
                f0 = fb[0][pl.ds(sv, 16)]
                f1 = fb[1][pl.ds(sv, 16)]
                g0 = 1.0 - f0
                g1 = 1.0 - f1
                w = (g0 * g1, g0 * f1, f0 * g1, f0 * f1)
                a0 = (w[0] * plsc.load_gather(rb[0], [pidx, cc[0]])
                      + w[1] * plsc.load_gather(rb[1], [pidx, cc[1]])
                      + w[2] * plsc.load_gather(rb[2], [pidx, cc[2]])
                      + w[3] * plsc.load_gather(rb[3], [pidx, cc[3]]))
                a1 = (w[0] * plsc.load_gather(rb[0], [pidx, cc[0] + 8])
                      + w[1] * plsc.load_gather(rb[1], [pidx, cc[1] + 8])
                      + w[2] * plsc.load_gather(rb[2], [pidx, cc[2] + 8])
                      + w[3] * plsc.load_gather(rb[3], [pidx, cc[3] + 8]))
                eidx = pidx * _ECP
                plsc.store_scatter(encb, [eidx + 2 * l], a0)
                plsc.store_scatter(encb, [eidx + (2 * l + 1)], a1)
                return c
            lax.fori_loop(0, ng, pc, 0)

        def sub_body(sub, carry):
            gbase = wid * pw + sub * b
            pltpu.sync_copy(x0_hbm.at[pl.ds(gbase, b)], x0b)
            pltpu.sync_copy(x1_hbm.at[pl.ds(gbase, b)], x1b)

            def pz(j, c):
                eidx = (j * 16 + io) * _ECP
                z16 = jnp.zeros((16,), jnp.float32)
                plsc.store_scatter(encb, [eidx + _EC], z16)
                plsc.store_scatter(encb, [eidx + (_EC + 1)], z16)
                return c
            lax.fori_loop(0, ng, pz, 0)

            do_pa(0, sets[0])
            cps = do_fire(sets[0])
            for l in range(1, _L):
                cur = sets[l & 1]
                do_pa(l, cur)
                ncps = do_fire(cur)
                for cp in cps:
                    cp.wait()
                do_pc(l - 1, sets[(l - 1) & 1])
                cps = ncps
            for cp in cps:
                cp.wait()
            do_pc(_L - 1, sets[(_L - 1) & 1])

            pltpu.sync_copy(encb, out_hbm.at[pl.ds(gbase * _ECP, b * _ECP)])
            return carry

        lax.fori_loop(0, nsub, sub_body, 0)

    return encode


_encode = _build_encode(_N, 512)

_RPK_BQ = 512
_RPK_R = _L * _T // 64


def _repack_body(in_ref, out_ref):
    x = in_ref[...]
    xn = pltpu.roll(x, _RPK_BQ - 1, 0)
    zlo = jnp.concatenate(
        [p for k in range(8) for p in (x[:, 8*k:8*k+8], xn[:, 8*k:8*k+8])],
        axis=1)
    zhi = jnp.concatenate(
        [p for k in range(8, 16) for p in (x[:, 8*k:8*k+8], xn[:, 8*k:8*k+8])],
        axis=1)
    zhr = pltpu.roll(zhi, 1, 0)
    rows = lax.broadcasted_iota(jnp.int32, (_RPK_BQ, 128), 0)
    out_ref[...] = jnp.where((rows & 1) == 0, zlo, zhr)


def _repack(a):
    return pl.pallas_call(
        _repack_body,
        grid=(_RPK_R // _RPK_BQ,),
        in_specs=[pl.BlockSpec((_RPK_BQ, 128), lambda i: (i, 0))],
        out_specs=pl.BlockSpec((_RPK_BQ, 128), lambda i: (i, 0)),
        out_shape=jax.ShapeDtypeStruct((_RPK_R, 128), jnp.float32),
    )(a)


_MLP_NB = 2048


def _mlp_body(enc_ref, w1_ref, w2_ref, w3_ref, out_ref):
    h = jnp.maximum(
        jnp.dot(enc_ref[...], w1_ref[...], preferred_element_type=jnp.float32), 0.0)
    h = jnp.maximum(
        jnp.dot(h, w2_ref[...], preferred_element_type=jnp.float32), 0.0)
    out_ref[...] = jnp.dot(h, w3_ref[...], preferred_element_type=jnp.float32)


def _mlp(enc, w1p, w2, w3):
    n = enc.shape[0]
    return pl.pallas_call(
        _mlp_body,
        grid=(n // _MLP_NB,),
        in_specs=[
            pl.BlockSpec((_MLP_NB, _ECP), lambda i: (i, 0)),
            pl.BlockSpec((_ECP, 64), lambda i: (0, 0)),
            pl.BlockSpec((64, 64), lambda i: (0, 0)),
            pl.BlockSpec((64, 3), lambda i: (0, 0)),
        ],
        out_specs=pl.BlockSpec((_MLP_NB, 3), lambda i: (i, 0)),
        out_shape=jax.ShapeDtypeStruct((n, 3), jnp.float32),
    )(enc, w1p, w2, w3)


def kernel(x, grid, W1, W2, W3):
    x0 = jnp.asarray(x[:, 0])
    x1 = jnp.asarray(x[:, 1])
    a3 = (grid.reshape(_L, _T // 128, 128, _F)
          .transpose(0, 1, 3, 2)
          .reshape(_RPK_R, 128))
    gridf = _repack(a3).reshape(_L * _T // 8, 16)
    enc = _encode(x0, x1, gridf).reshape(_N, _ECP)
    w1p = jnp.concatenate([W1, jnp.zeros((2, 64), jnp.float32)], axis=0)
    return _mlp(enc, w1p, W2, W3)
